# Initial kernel scaffold; baseline (speedup 1.0000x reference)
import jax
import jax.numpy as jnp
from jax.experimental import pallas as pl


def kernel(node_type, pos_undirected, edge_index, ast_table, deg_table, gin_w1, gin_b1, gin_w2, gin_b2, pred_w, pred_b):
    raise NotImplementedError("write your pallas kernel here")


# speedup vs baseline: 2.1103x; 2.1103x over previous
import functools

import jax
import jax.numpy as jnp
from jax import lax
from jax.experimental import pallas as pl
from jax.experimental.pallas import tpu as pltpu
from jax.experimental.pallas import tpu_sc as plsc

N = 50000
E = 800000
HID = 128
OUT = 128
NLAYERS = 5
GW = 32
NCHUNK = 128
NODE_CHUNKS = N // NCHUNK
NODE_TAIL = N - NODE_CHUNKS * NCHUNK
EPT = E // 16
EDGE_CHUNKS = EPT // NCHUNK
EDGE_TAIL = EPT - EDGE_CHUNKS * NCHUNK

_MESH = plsc.VectorSubcoreMesh(core_axis_name="c", subcore_axis_name="s")


def _zero_vmem_2d(ref, rows, width):
    z = jnp.zeros((16,), ref.dtype)

    def body(i, _):
        for j in range(width // 16):
            ref[i, pl.ds(j * 16, 16)] = z
        return 0

    lax.fori_loop(0, rows, body, 0)


def _zero_vmem_1d(ref, n):
    z = jnp.zeros((16,), ref.dtype)

    def body(i, _):
        ref[pl.ds(i * 16, 16)] = z
        return 0

    lax.fori_loop(0, n // 16, body, 0)


CROWS = 400


def _phase_a_body(dst_e_ref, ntype_ref, ast_tab_ref, deg_tab_ref,
                  ast_emb_ref, deg_emb_ref,
                  spmem_cnt, cnt2, dstbuf, z2, idr, cbuf128, idx128, rows128,
                  cbuf80, idx80, rows80, sem):
    c = lax.axis_index("c")
    s = lax.axis_index("s")

    @pl.when(c == 0)
    def _sc0():
        _zero_vmem_2d(z2, CROWS // 16, NCHUNK)
        _zero_vmem_2d(cnt2, CROWS, NCHUNK)
        pltpu.sync_copy(z2, spmem_cnt.at[pl.ds(s * (CROWS // 16), CROWS // 16)])

        ones16 = jnp.ones((16,), jnp.int32)
        ebase = s * EPT

        def cbody(j, _):
            pltpu.sync_copy(dst_e_ref.at[pl.ds(ebase + j * 2000, 2000)], dstbuf)

            def ibody(k, _):
                v = dstbuf[pl.ds(k * 16, 16)]
                hi = lax.shift_right_logical(v, 7)
                lo = jnp.bitwise_and(v, 127)
                plsc.addupdate_scatter(cnt2, [hi, lo], ones16)
                return 0

            lax.fori_loop(0, 125, ibody, 0)
            return 0

        lax.fori_loop(0, EPT // 2000, cbody, 0)

        plsc.subcore_barrier()
        iota16 = lax.iota(jnp.int32, 16)
        for b in range(5):
            for k in range(5):
                idr[pl.ds(k * 16, 16)] = iota16 + (b * 80 + k * 16)
            pltpu.sync_copy(cnt2.at[pl.ds(b * 80, 80)], spmem_cnt.at[idr],
                            add=True)
        plsc.subcore_barrier()

        def gbody(j, _):
            g = j * 16 + s

            @pl.when(g < NODE_CHUNKS)
            def _():
                base = g * NCHUNK
                pltpu.sync_copy(spmem_cnt.at[g], cbuf128)
                for k in range(NCHUNK // 16):
                    idx128[pl.ds(k * 16, 16)] = jnp.minimum(cbuf128[pl.ds(k * 16, 16)], 512)
                pltpu.async_copy(deg_tab_ref.at[idx128], rows128, sem).wait()
                pltpu.sync_copy(rows128, deg_emb_ref.at[pl.ds(base, NCHUNK)])

            @pl.when(g == NODE_CHUNKS)
            def _():
                base = NODE_CHUNKS * NCHUNK
                pltpu.sync_copy(spmem_cnt.at[NODE_CHUNKS, pl.ds(0, NODE_TAIL)], cbuf80)
                for k in range(NODE_TAIL // 16):
                    idx80[pl.ds(k * 16, 16)] = jnp.minimum(cbuf80[pl.ds(k * 16, 16)], 512)
                pltpu.async_copy(deg_tab_ref.at[idx80], rows80, sem).wait()
                pltpu.sync_copy(rows80, deg_emb_ref.at[pl.ds(base, NODE_TAIL)])
            return 0

        lax.fori_loop(0, 25, gbody, 0)

    @pl.when(c == 1)
    def _sc1():
        def gbody(j, _):
            g = j * 16 + s

            @pl.when(g < NODE_CHUNKS)
            def _():
                base = g * NCHUNK
                pltpu.sync_copy(ntype_ref.at[pl.ds(base, NCHUNK)], idx128)
                pltpu.async_copy(ast_tab_ref.at[idx128], rows128, sem).wait()
                pltpu.sync_copy(rows128, ast_emb_ref.at[pl.ds(base, NCHUNK)])

            @pl.when(g == NODE_CHUNKS)
            def _():
                base = NODE_CHUNKS * NCHUNK
                pltpu.sync_copy(ntype_ref.at[pl.ds(base, NODE_TAIL)], idx80)
                pltpu.async_copy(ast_tab_ref.at[idx80], rows80, sem).wait()
                pltpu.sync_copy(rows80, ast_emb_ref.at[pl.ds(base, NODE_TAIL)])
            return 0

        lax.fori_loop(0, 25, gbody, 0)


def _make_phase_a():
    return pl.kernel(
        _phase_a_body,
        out_type=(jax.ShapeDtypeStruct((N, GW), jnp.float32),
                  jax.ShapeDtypeStruct((N, GW), jnp.float32)),
        mesh=_MESH,
        compiler_params=pltpu.CompilerParams(
            use_tc_tiling_on_sc=False, needs_layout_passes=False),
        scratch_types=[
            pltpu.VMEM_SHARED((CROWS, NCHUNK), jnp.int32),
            pltpu.VMEM((CROWS, NCHUNK), jnp.int32),
            pltpu.VMEM((2000,), jnp.int32),
            pltpu.VMEM((CROWS // 16, NCHUNK), jnp.int32),
            pltpu.VMEM((80,), jnp.int32),
            pltpu.VMEM((NCHUNK,), jnp.int32),
            pltpu.VMEM((NCHUNK,), jnp.int32),
            pltpu.VMEM((NCHUNK, GW), jnp.float32),
            pltpu.VMEM((NODE_TAIL,), jnp.int32),
            pltpu.VMEM((NODE_TAIL,), jnp.int32),
            pltpu.VMEM((NODE_TAIL, GW), jnp.float32),
            pltpu.SemaphoreType.DMA,
        ],
    )


def _seg_group(h_ref, agg_ref, src_e_ref, dst_e_ref, spmem, zrows,
               src128, dst128, rows128, src80, dst80, rows80, sem, s):
    def zbody(j, _):
        g = j * 16 + s

        @pl.when(g < NODE_CHUNKS)
        def _():
            pltpu.sync_copy(zrows, spmem.at[pl.ds(g * NCHUNK, NCHUNK)])

        @pl.when(g == NODE_CHUNKS)
        def _():
            pltpu.sync_copy(zrows.at[pl.ds(0, NODE_TAIL)],
                            spmem.at[pl.ds(NODE_CHUNKS * NCHUNK, NODE_TAIL)])
        return 0

    lax.fori_loop(0, 25, zbody, 0)
    plsc.subcore_barrier()

    ebase = s * EPT

    def ebody(j, _):
        b = ebase + j * NCHUNK
        pltpu.sync_copy(src_e_ref.at[pl.ds(b, NCHUNK)], src128)
        pltpu.sync_copy(dst_e_ref.at[pl.ds(b, NCHUNK)], dst128)
        pltpu.async_copy(h_ref.at[src128], rows128, sem).wait()
        pltpu.sync_copy(rows128, spmem.at[dst128], add=True)
        return 0

    lax.fori_loop(0, EDGE_CHUNKS, ebody, 0)
    b = ebase + EDGE_CHUNKS * NCHUNK
    pltpu.sync_copy(src_e_ref.at[pl.ds(b, EDGE_TAIL)], src80)
    pltpu.sync_copy(dst_e_ref.at[pl.ds(b, EDGE_TAIL)], dst80)
    pltpu.async_copy(h_ref.at[src80], rows80, sem).wait()
    pltpu.sync_copy(rows80, spmem.at[dst80], add=True)

    plsc.subcore_barrier()

    def dbody(j, _):
        g = j * 16 + s

        @pl.when(g < NODE_CHUNKS)
        def _():
            base = g * NCHUNK
            pltpu.sync_copy(spmem.at[pl.ds(base, NCHUNK)], agg_ref.at[pl.ds(base, NCHUNK)])

        @pl.when(g == NODE_CHUNKS)
        def _():
            base = NODE_CHUNKS * NCHUNK
            pltpu.sync_copy(spmem.at[pl.ds(base, NODE_TAIL)], agg_ref.at[pl.ds(base, NODE_TAIL)])
        return 0

    lax.fori_loop(0, 25, dbody, 0)
    plsc.subcore_barrier()


@functools.cache
def _make_segsum(ngroups):
    def body(*refs):
        h_refs = refs[:ngroups]
        src_e_ref = refs[ngroups]
        dst_e_ref = refs[ngroups + 1]
        agg_refs = refs[ngroups + 2:2 * ngroups + 2]
        (spmem, zrows, src128, dst128, rows128,
         src80, dst80, rows80, sem) = refs[2 * ngroups + 2:]
        c = lax.axis_index("c")
        s = lax.axis_index("s")
        _zero_vmem_2d(zrows, NCHUNK, GW)

        sc0_groups = [g for g in range(ngroups) if g % 2 == 0]
        sc1_groups = [g for g in range(ngroups) if g % 2 == 1]

        @pl.when(c == 0)
        def _():
            for g in sc0_groups:
                _seg_group(h_refs[g], agg_refs[g], src_e_ref, dst_e_ref,
                           spmem, zrows, src128, dst128, rows128,
                           src80, dst80, rows80, sem, s)

        @pl.when(c == 1)
        def _():
            for g in sc1_groups:
                _seg_group(h_refs[g], agg_refs[g], src_e_ref, dst_e_ref,
                           spmem, zrows, src128, dst128, rows128,
                           src80, dst80, rows80, sem, s)

    return pl.kernel(
        body,
        out_type=tuple(jax.ShapeDtypeStruct((N, GW), jnp.float32)
                       for _ in range(ngroups)),
        mesh=_MESH,
        compiler_params=pltpu.CompilerParams(
            use_tc_tiling_on_sc=False, needs_layout_passes=False),
        scratch_types=[
            pltpu.VMEM_SHARED((N, GW), jnp.float32),
            pltpu.VMEM((NCHUNK, GW), jnp.float32),
            pltpu.VMEM((NCHUNK,), jnp.int32),
            pltpu.VMEM((NCHUNK,), jnp.int32),
            pltpu.VMEM((NCHUNK, GW), jnp.float32),
            pltpu.VMEM((EDGE_TAIL,), jnp.int32),
            pltpu.VMEM((EDGE_TAIL,), jnp.int32),
            pltpu.VMEM((EDGE_TAIL, GW), jnp.float32),
            pltpu.SemaphoreType.DMA,
        ],
    )


BN = 2000
GSTEPS = N // BN


def _dot(a, b):
    return lax.dot_general(a, b, (((1,), (0,)), ((), ())),
                           preferred_element_type=jnp.float32,
                           precision=lax.Precision.HIGHEST)


@functools.cache
def _make_mlp(ngroups, first):
    def body(*refs):
        h = refs[:ngroups]
        a = refs[ngroups:2 * ngroups]
        w1, b1, w2, b2 = refs[2 * ngroups:2 * ngroups + 4]
        outs = refs[2 * ngroups + 4:]
        ho = outs[:4]
        pooled_out = outs[4]
        i = pl.program_id(0)

        x = jnp.concatenate([h[g][...] + a[g][...] for g in range(ngroups)],
                            axis=1)
        y = jnp.maximum(_dot(x, w1[...]) + b1[...], 0.0)
        z = jnp.maximum(_dot(y, w2[...]) + b2[...], 0.0)
        for g in range(4):
            ho[g][...] = z[:, g * GW:(g + 1) * GW]

        @pl.when(i == 0)
        def _():
            pooled_out[...] = jnp.zeros_like(pooled_out)
            if first:
                outs[5][...] = jnp.zeros_like(outs[5])

        pooled_out[...] += jnp.sum(z, axis=0, keepdims=True)
        if first:
            pin = jnp.concatenate(
                [jnp.sum(h[g][...], axis=0, keepdims=True) for g in range(ngroups)],
                axis=1)
            outs[5][...] += pin

    din = GW * ngroups
    out_shape = [jax.ShapeDtypeStruct((N, GW), jnp.float32) for _ in range(4)]
    out_specs = [pl.BlockSpec((BN, GW), lambda i: (i, 0)) for _ in range(4)]
    out_shape.append(jax.ShapeDtypeStruct((1, HID), jnp.float32))
    out_specs.append(pl.BlockSpec((1, HID), lambda i: (0, 0)))
    if first:
        out_shape.append(jax.ShapeDtypeStruct((1, din), jnp.float32))
        out_specs.append(pl.BlockSpec((1, din), lambda i: (0, 0)))

    in_specs = ([pl.BlockSpec((BN, GW), lambda i: (i, 0)) for _ in range(2 * ngroups)]
                + [pl.BlockSpec((din, HID), lambda i: (0, 0)),
                   pl.BlockSpec((1, HID), lambda i: (0, 0)),
                   pl.BlockSpec((HID, HID), lambda i: (0, 0)),
                   pl.BlockSpec((1, HID), lambda i: (0, 0))])

    return pl.pallas_call(
        body,
        grid=(GSTEPS,),
        in_specs=in_specs,
        out_specs=out_specs,
        out_shape=out_shape,
        compiler_params=pltpu.CompilerParams(
            dimension_semantics=("arbitrary",)),
    )


def _readout_body(p0, p1, p2, p3, p4, p5, w0, w1, w2, w3, w4, w5, bsum, out):
    acc = _dot(p0[...], w0[...])
    for p, w in ((p1, w1), (p2, w2), (p3, w3), (p4, w4), (p5, w5)):
        acc = acc + _dot(p[...], w[...])
    out[...] = acc + bsum[...]


@functools.cache
def _make_readout(din0):
    return pl.pallas_call(
        _readout_body,
        out_shape=jax.ShapeDtypeStruct((1, OUT), jnp.float32),
    )


def kernel(node_type, pos_undirected, edge_index, ast_table, deg_table,
           gin_w1, gin_b1, gin_w2, gin_b2, pred_w, pred_b):
    f32 = jnp.float32
    deg_table32 = jnp.pad(deg_table, ((0, 0), (0, GW - deg_table.shape[1])))

    src_e = edge_index[0]
    dst_e = edge_index[1]
    ast_emb, deg_emb = _make_phase_a()(dst_e, node_type, ast_table,
                                       deg_table32)

    groups = (pos_undirected, ast_emb, deg_emb)
    aggs = _make_segsum(3)(*groups, src_e, dst_e)
    w1p = jnp.concatenate([gin_w1[0][0:32], gin_w1[0][48:80],
                           gin_w1[0][32:48], jnp.zeros((16, HID), f32)], axis=0)
    outs = _make_mlp(3, True)(*groups, *aggs, w1p, gin_b1[0].reshape(1, HID),
                              gin_w2[0], gin_b2[0].reshape(1, HID))
    h_groups = outs[:4]
    pooled = [outs[5], outs[4]]

    for i in range(1, NLAYERS):
        aggs = _make_segsum(4)(*h_groups, src_e, dst_e)
        outs = _make_mlp(4, False)(*h_groups, *aggs, gin_w1[i],
                                   gin_b1[i].reshape(1, HID), gin_w2[i],
                                   gin_b2[i].reshape(1, HID))
        h_groups = outs[:4]
        pooled.append(outs[4])

    pw0 = jnp.concatenate([pred_w[0][0:32], pred_w[0][48:80],
                           pred_w[0][32:48], jnp.zeros((16, OUT), f32)], axis=0)
    bsum = sum(pred_b[1:], pred_b[0]).reshape(1, OUT)
    score = _make_readout(pw0.shape[0])(
        pooled[0], pooled[1], pooled[2], pooled[3], pooled[4], pooled[5],
        pw0, pred_w[1], pred_w[2], pred_w[3], pred_w[4], pred_w[5], bsum)
    return score


# speedup vs baseline: 6.9011x; 3.2703x over previous
import functools

import jax
import jax.numpy as jnp
from jax import lax
from jax.experimental import pallas as pl
from jax.experimental.pallas import tpu as pltpu
from jax.experimental.pallas import tpu_sc as plsc

N = 50000
E = 800000
HID = 128
OUT = 128
NLAYERS = 5
GW = 32
NCHUNK = 128
NODE_CHUNKS = N // NCHUNK
NODE_TAIL = N - NODE_CHUNKS * NCHUNK
EPT = E // 16
EDGE_CHUNKS = EPT // NCHUNK
EDGE_TAIL = EPT - EDGE_CHUNKS * NCHUNK

_MESH = plsc.VectorSubcoreMesh(core_axis_name="c", subcore_axis_name="s")


def _zero_vmem_2d(ref, rows, width):
    z = jnp.zeros((16,), ref.dtype)

    def body(i, _):
        for j in range(width // 16):
            ref[i, pl.ds(j * 16, 16)] = z
        return 0

    lax.fori_loop(0, rows, body, 0)


def _zero_vmem_1d(ref, n):
    z = jnp.zeros((16,), ref.dtype)

    def body(i, _):
        ref[pl.ds(i * 16, 16)] = z
        return 0

    lax.fori_loop(0, n // 16, body, 0)


CROWS = 400


def _phase_a_body(dst_e_ref, ntype_ref, ast_tab_ref, deg_tab_ref,
                  ast_emb_ref, deg_emb_ref,
                  spmem_cnt, cnt2, dstbuf, z2, idr, cbuf128, idx128, rows128,
                  cbuf80, idx80, rows80, sem):
    c = lax.axis_index("c")
    s = lax.axis_index("s")

    @pl.when(c == 0)
    def _sc0():
        _zero_vmem_2d(z2, CROWS // 16, NCHUNK)
        _zero_vmem_2d(cnt2, CROWS, NCHUNK)
        pltpu.sync_copy(z2, spmem_cnt.at[pl.ds(s * (CROWS // 16), CROWS // 16)])

        ones16 = jnp.ones((16,), jnp.int32)
        ebase = s * EPT

        def cbody(j, _):
            pltpu.sync_copy(dst_e_ref.at[pl.ds(ebase + j * 2000, 2000)], dstbuf)

            def ibody(k, _):
                v = dstbuf[pl.ds(k * 16, 16)]
                hi = lax.shift_right_logical(v, 7)
                lo = jnp.bitwise_and(v, 127)
                plsc.addupdate_scatter(cnt2, [hi, lo], ones16)
                return 0

            lax.fori_loop(0, 125, ibody, 0)
            return 0

        lax.fori_loop(0, EPT // 2000, cbody, 0)

        plsc.subcore_barrier()
        iota16 = lax.iota(jnp.int32, 16)
        for b in range(5):
            for k in range(5):
                idr[pl.ds(k * 16, 16)] = iota16 + (b * 80 + k * 16)
            pltpu.sync_copy(cnt2.at[pl.ds(b * 80, 80)], spmem_cnt.at[idr],
                            add=True)
        plsc.subcore_barrier()

        def gbody(j, _):
            g = j * 16 + s

            @pl.when(g < NODE_CHUNKS)
            def _():
                base = g * NCHUNK
                pltpu.sync_copy(spmem_cnt.at[g], cbuf128)
                for k in range(NCHUNK // 16):
                    idx128[pl.ds(k * 16, 16)] = jnp.minimum(cbuf128[pl.ds(k * 16, 16)], 512)
                pltpu.async_copy(deg_tab_ref.at[idx128], rows128, sem).wait()
                pltpu.sync_copy(rows128, deg_emb_ref.at[pl.ds(base, NCHUNK)])

            @pl.when(g == NODE_CHUNKS)
            def _():
                base = NODE_CHUNKS * NCHUNK
                pltpu.sync_copy(spmem_cnt.at[NODE_CHUNKS, pl.ds(0, NODE_TAIL)], cbuf80)
                for k in range(NODE_TAIL // 16):
                    idx80[pl.ds(k * 16, 16)] = jnp.minimum(cbuf80[pl.ds(k * 16, 16)], 512)
                pltpu.async_copy(deg_tab_ref.at[idx80], rows80, sem).wait()
                pltpu.sync_copy(rows80, deg_emb_ref.at[pl.ds(base, NODE_TAIL)])
            return 0

        lax.fori_loop(0, 25, gbody, 0)

    @pl.when(c == 1)
    def _sc1():
        def gbody(j, _):
            g = j * 16 + s

            @pl.when(g < NODE_CHUNKS)
            def _():
                base = g * NCHUNK
                pltpu.sync_copy(ntype_ref.at[pl.ds(base, NCHUNK)], idx128)
                pltpu.async_copy(ast_tab_ref.at[idx128], rows128, sem).wait()
                pltpu.sync_copy(rows128, ast_emb_ref.at[pl.ds(base, NCHUNK)])

            @pl.when(g == NODE_CHUNKS)
            def _():
                base = NODE_CHUNKS * NCHUNK
                pltpu.sync_copy(ntype_ref.at[pl.ds(base, NODE_TAIL)], idx80)
                pltpu.async_copy(ast_tab_ref.at[idx80], rows80, sem).wait()
                pltpu.sync_copy(rows80, ast_emb_ref.at[pl.ds(base, NODE_TAIL)])
            return 0

        lax.fori_loop(0, 25, gbody, 0)


def _make_phase_a():
    return pl.kernel(
        _phase_a_body,
        out_type=(jax.ShapeDtypeStruct((N, GW), jnp.float32),
                  jax.ShapeDtypeStruct((N, GW), jnp.float32)),
        mesh=_MESH,
        compiler_params=pltpu.CompilerParams(
            use_tc_tiling_on_sc=False, needs_layout_passes=False),
        scratch_types=[
            pltpu.VMEM_SHARED((CROWS, NCHUNK), jnp.int32),
            pltpu.VMEM((CROWS, NCHUNK), jnp.int32),
            pltpu.VMEM((2000,), jnp.int32),
            pltpu.VMEM((CROWS // 16, NCHUNK), jnp.int32),
            pltpu.VMEM((80,), jnp.int32),
            pltpu.VMEM((NCHUNK,), jnp.int32),
            pltpu.VMEM((NCHUNK,), jnp.int32),
            pltpu.VMEM((NCHUNK, GW), jnp.float32),
            pltpu.VMEM((NODE_TAIL,), jnp.int32),
            pltpu.VMEM((NODE_TAIL,), jnp.int32),
            pltpu.VMEM((NODE_TAIL, GW), jnp.float32),
            pltpu.SemaphoreType.DMA,
        ],
    )


def _seg_group(h_ref, agg_ref, src_e_ref, dst_e_ref, spmem,
               si, di, si_t, di_t, rows4, isems, gsems, dsem, s):
    nbase = s * (N // 16)
    ebase = s * EPT
    _zero_vmem_2d(rows4[0], 125, GW)
    zsrc = rows4[0].at[pl.ds(0, 125)]
    zd = [pltpu.async_copy(zsrc, spmem.at[pl.ds(nbase + j * 125, 125)], dsem)
          for j in range(25)]
    for d in zd:
        d.wait()
    plsc.subcore_barrier()

    def load_idx(c, r8):
        pltpu.async_copy(src_e_ref.at[pl.ds(ebase + c * NCHUNK, NCHUNK)],
                         si[r8], isems[r8])
        pltpu.async_copy(dst_e_ref.at[pl.ds(ebase + c * NCHUNK, NCHUNK)],
                         di[r8], isems[r8])

    def wait_idx(r8):
        pltpu.make_async_copy(src_e_ref.at[pl.ds(0, NCHUNK)], si[r8],
                              isems[r8]).wait()
        pltpu.make_async_copy(dst_e_ref.at[pl.ds(0, NCHUNK)], di[r8],
                              isems[r8]).wait()

    def gather(r8, r4):
        pltpu.async_copy(h_ref.at[si[r8]], rows4[r4], gsems[r4])

    def wait_gather(r4):
        pltpu.make_async_copy(h_ref.at[pl.ds(0, NCHUNK)], rows4[r4],
                              gsems[r4]).wait()

    def scatter(r8, r4):
        pltpu.sync_copy(rows4[r4], spmem.at[di[r8]], add=True)

    for cc in range(8):
        load_idx(cc, cc)
    for cc in range(4):
        wait_idx(cc)
        gather(cc, cc)

    def round_body(k, _):
        for rr in range(8):
            c = 8 * k + rr
            r4 = rr % 4
            wait_gather(r4)
            scatter(rr, r4)

            @pl.when(c < EDGE_CHUNKS - 8)
            def _():
                load_idx(c + 8, rr)
            wait_idx((rr + 4) % 8)
            gather((rr + 4) % 8, r4)
        return 0

    lax.fori_loop(0, EDGE_CHUNKS // 8, round_body, 0)
    for cc in range(EDGE_CHUNKS // 8 * 8, EDGE_CHUNKS):
        r8 = cc % 8
        r4 = cc % 4
        if r8 >= 4:
            wait_idx(r8)
            gather(r8, r4)
        wait_gather(r4)
        scatter(r8, r4)
    b = ebase + EDGE_CHUNKS * NCHUNK
    pltpu.sync_copy(src_e_ref.at[pl.ds(b, EDGE_TAIL)], si_t)
    pltpu.sync_copy(dst_e_ref.at[pl.ds(b, EDGE_TAIL)], di_t)
    pltpu.async_copy(h_ref.at[si_t], rows4[0].at[pl.ds(0, EDGE_TAIL)],
                     gsems[0]).wait()
    pltpu.sync_copy(rows4[0].at[pl.ds(0, EDGE_TAIL)], spmem.at[di_t],
                    add=True)

    plsc.subcore_barrier()
    pltpu.sync_copy(spmem.at[pl.ds(nbase, N // 16)],
                    agg_ref.at[pl.ds(nbase, N // 16)])
    plsc.subcore_barrier()


@functools.cache
def _make_segsum(ngroups):
    def body(*refs):
        h_refs = refs[:ngroups]
        src_e_ref = refs[ngroups]
        dst_e_ref = refs[ngroups + 1]
        agg_refs = refs[ngroups + 2:2 * ngroups + 2]
        rest = refs[2 * ngroups + 2:]
        spmem = rest[0]
        si = rest[1:9]
        di = rest[9:17]
        si_t, di_t = rest[17], rest[18]
        rows4 = rest[19:23]
        isems = rest[23:31]
        gsems = rest[31:35]
        dsem = rest[35]
        c = lax.axis_index("c")
        s = lax.axis_index("s")

        sc0_groups = [g for g in range(ngroups) if g % 2 == 0]
        sc1_groups = [g for g in range(ngroups) if g % 2 == 1]

        @pl.when(c == 0)
        def _():
            for g in sc0_groups:
                _seg_group(h_refs[g], agg_refs[g], src_e_ref, dst_e_ref,
                           spmem, si, di, si_t, di_t, rows4, isems, gsems,
                           dsem, s)

        @pl.when(c == 1)
        def _():
            for g in sc1_groups:
                _seg_group(h_refs[g], agg_refs[g], src_e_ref, dst_e_ref,
                           spmem, si, di, si_t, di_t, rows4, isems, gsems,
                           dsem, s)

    return pl.kernel(
        body,
        out_type=tuple(jax.ShapeDtypeStruct((N, GW), jnp.float32)
                       for _ in range(ngroups)),
        mesh=_MESH,
        compiler_params=pltpu.CompilerParams(
            use_tc_tiling_on_sc=False, needs_layout_passes=False),
        scratch_types=(
            [pltpu.VMEM_SHARED((N, GW), jnp.float32)]
            + [pltpu.VMEM((NCHUNK,), jnp.int32) for _ in range(16)]
            + [pltpu.VMEM((EDGE_TAIL,), jnp.int32) for _ in range(2)]
            + [pltpu.VMEM((NCHUNK, GW), jnp.float32) for _ in range(4)]
            + [pltpu.SemaphoreType.DMA for _ in range(13)]
        ),
    )


BN = 2000
GSTEPS = N // BN


def _dot(a, b):
    return lax.dot_general(a, b, (((1,), (0,)), ((), ())),
                           preferred_element_type=jnp.float32,
                           precision=lax.Precision.HIGHEST)


@functools.cache
def _make_mlp(ngroups, first):
    def body(*refs):
        h = refs[:ngroups]
        a = refs[ngroups:2 * ngroups]
        w1, b1, w2, b2 = refs[2 * ngroups:2 * ngroups + 4]
        outs = refs[2 * ngroups + 4:]
        ho = outs[:4]
        pooled_out = outs[4]
        i = pl.program_id(0)

        x = jnp.concatenate([h[g][...] + a[g][...] for g in range(ngroups)],
                            axis=1)
        y = jnp.maximum(_dot(x, w1[...]) + b1[...], 0.0)
        z = jnp.maximum(_dot(y, w2[...]) + b2[...], 0.0)
        for g in range(4):
            ho[g][...] = z[:, g * GW:(g + 1) * GW]

        @pl.when(i == 0)
        def _():
            pooled_out[...] = jnp.zeros_like(pooled_out)
            if first:
                outs[5][...] = jnp.zeros_like(outs[5])

        pooled_out[...] += jnp.sum(z, axis=0, keepdims=True)
        if first:
            pin = jnp.concatenate(
                [jnp.sum(h[g][...], axis=0, keepdims=True) for g in range(ngroups)],
                axis=1)
            outs[5][...] += pin

    din = GW * ngroups
    out_shape = [jax.ShapeDtypeStruct((N, GW), jnp.float32) for _ in range(4)]
    out_specs = [pl.BlockSpec((BN, GW), lambda i: (i, 0)) for _ in range(4)]
    out_shape.append(jax.ShapeDtypeStruct((1, HID), jnp.float32))
    out_specs.append(pl.BlockSpec((1, HID), lambda i: (0, 0)))
    if first:
        out_shape.append(jax.ShapeDtypeStruct((1, din), jnp.float32))
        out_specs.append(pl.BlockSpec((1, din), lambda i: (0, 0)))

    in_specs = ([pl.BlockSpec((BN, GW), lambda i: (i, 0)) for _ in range(2 * ngroups)]
                + [pl.BlockSpec((din, HID), lambda i: (0, 0)),
                   pl.BlockSpec((1, HID), lambda i: (0, 0)),
                   pl.BlockSpec((HID, HID), lambda i: (0, 0)),
                   pl.BlockSpec((1, HID), lambda i: (0, 0))])

    return pl.pallas_call(
        body,
        grid=(GSTEPS,),
        in_specs=in_specs,
        out_specs=out_specs,
        out_shape=out_shape,
        compiler_params=pltpu.CompilerParams(
            dimension_semantics=("arbitrary",)),
    )


def _readout_body(p0, p1, p2, p3, p4, p5, w0, w1, w2, w3, w4, w5, bsum, out):
    acc = _dot(p0[...], w0[...])
    for p, w in ((p1, w1), (p2, w2), (p3, w3), (p4, w4), (p5, w5)):
        acc = acc + _dot(p[...], w[...])
    out[...] = acc + bsum[...]


@functools.cache
def _make_readout(din0):
    return pl.pallas_call(
        _readout_body,
        out_shape=jax.ShapeDtypeStruct((1, OUT), jnp.float32),
    )


def kernel(node_type, pos_undirected, edge_index, ast_table, deg_table,
           gin_w1, gin_b1, gin_w2, gin_b2, pred_w, pred_b):
    f32 = jnp.float32
    deg_table32 = jnp.pad(deg_table, ((0, 0), (0, GW - deg_table.shape[1])))

    src_e = edge_index[0]
    dst_e = edge_index[1]
    ast_emb, deg_emb = _make_phase_a()(dst_e, node_type, ast_table,
                                       deg_table32)

    groups = (pos_undirected, ast_emb, deg_emb)
    aggs = _make_segsum(3)(*groups, src_e, dst_e)
    w1p = jnp.concatenate([gin_w1[0][0:32], gin_w1[0][48:80],
                           gin_w1[0][32:48], jnp.zeros((16, HID), f32)], axis=0)
    outs = _make_mlp(3, True)(*groups, *aggs, w1p, gin_b1[0].reshape(1, HID),
                              gin_w2[0], gin_b2[0].reshape(1, HID))
    h_groups = outs[:4]
    pooled = [outs[5], outs[4]]

    for i in range(1, NLAYERS):
        aggs = _make_segsum(4)(*h_groups, src_e, dst_e)
        outs = _make_mlp(4, False)(*h_groups, *aggs, gin_w1[i],
                                   gin_b1[i].reshape(1, HID), gin_w2[i],
                                   gin_b2[i].reshape(1, HID))
        h_groups = outs[:4]
        pooled.append(outs[4])

    pw0 = jnp.concatenate([pred_w[0][0:32], pred_w[0][48:80],
                           pred_w[0][32:48], jnp.zeros((16, OUT), f32)], axis=0)
    bsum = sum(pred_b[1:], pred_b[0]).reshape(1, OUT)
    score = _make_readout(pw0.shape[0])(
        pooled[0], pooled[1], pooled[2], pooled[3], pooled[4], pooled[5],
        pw0, pred_w[1], pred_w[2], pred_w[3], pred_w[4], pred_w[5], bsum)
    return score
